# baseline (speedup 1.0000x reference)
import functools

import jax
import jax.numpy as jnp
from jax import lax
from jax.experimental import pallas as pl
from jax.experimental.pallas import tpu as pltpu
from jax.experimental.pallas import tpu_sc as plsc

_NC = 2
_NS = 16
_NW = _NC * _NS
_LANES = 16
_COLS = 512
_R = 32
_UNROLL = 16


def _make_lut_kernel(n_rows: int):
    rows_per_w = n_rows // _NW
    n_chunks = rows_per_w // _R
    n_pairs = n_chunks // 2
    mesh = plsc.VectorSubcoreMesh(core_axis_name="c", subcore_axis_name="s")

    @functools.partial(
        pl.kernel,
        mesh=mesh,
        compiler_params=pltpu.CompilerParams(needs_layout_passes=False),
        out_type=jax.ShapeDtypeStruct((n_rows, _COLS), jnp.float32),
        scratch_types=[
            pltpu.VMEM((_LANES,), jnp.float32),
            pltpu.VMEM((_R, _COLS), jnp.int32),
            pltpu.VMEM((_R, _COLS), jnp.int32),
            pltpu.VMEM((_R, _COLS), jnp.float32),
            pltpu.VMEM((_R, _COLS), jnp.float32),
            pltpu.SemaphoreType.DMA,
            pltpu.SemaphoreType.DMA,
            pltpu.SemaphoreType.DMA,
            pltpu.SemaphoreType.DMA,
        ],
    )
    def lut_kernel(idx_hbm, tab_hbm, out_hbm, tab_v, idx0, idx1, o0, o1,
                   isem0, isem1, osem0, osem1):
        wid = lax.axis_index("s") * _NC + lax.axis_index("c")
        base = wid * rows_per_w
        pltpu.sync_copy(tab_hbm, tab_v)
        tab_vec = tab_v[...]

        def in_slice(g):
            return idx_hbm.at[pl.ds(base + g * _R, _R)]

        def out_slice(g):
            return out_hbm.at[pl.ds(base + g * _R, _R)]

        def compute(buf_i, buf_o):
            @plsc.parallel_loop(0, _R * _COLS, step=_LANES, unroll=_UNROLL)
            def _(o):
                r = o >> 9
                c = o & (_COLS - 1)
                iv = buf_i[r, pl.ds(c, _LANES)]
                buf_o[r, pl.ds(c, _LANES)] = jnp.take_along_axis(
                    tab_vec, iv, axis=0, mode="promise_in_bounds")

        def slot_step(h, g, buf_i, buf_o, isem, osem):
            pltpu.make_async_copy(in_slice(g), buf_i, isem).wait()

            @pl.when(h > 0)
            def _():
                pltpu.make_async_copy(buf_o, out_slice(g - 2), osem).wait()

            compute(buf_i, buf_o)
            pltpu.async_copy(buf_o, out_slice(g), osem)

            @pl.when(h < n_pairs - 1)
            def _():
                pltpu.async_copy(in_slice(g + 2), buf_i, isem)

        pltpu.async_copy(in_slice(0), idx0, isem0)
        pltpu.async_copy(in_slice(1), idx1, isem1)

        def pair(h, carry):
            g0 = 2 * h
            slot_step(h, g0, idx0, o0, isem0, osem0)
            slot_step(h, g0 + 1, idx1, o1, isem1, osem1)
            return carry

        lax.fori_loop(0, n_pairs, pair, 0)
        pltpu.make_async_copy(o0, out_slice(n_chunks - 2), osem0).wait()
        pltpu.make_async_copy(o1, out_slice(n_chunks - 1), osem1).wait()

    return lut_kernel


def kernel(shortest_path_len, spatial_embeddings):
    B, N, M = shortest_path_len.shape
    n_rows = B * N
    idx2d = shortest_path_len.reshape(n_rows, M).astype(jnp.int32)
    tab16 = jnp.pad(
        spatial_embeddings.reshape(-1).astype(jnp.float32),
        (0, _LANES - spatial_embeddings.shape[0]))
    out2d = _make_lut_kernel(n_rows)(idx2d, tab16)
    return out2d.reshape(B, N, M)


# speedup vs baseline: 1.0236x; 1.0236x over previous
import functools

import jax
import jax.numpy as jnp
from jax import lax
from jax.experimental import pallas as pl
from jax.experimental.pallas import tpu as pltpu
from jax.experimental.pallas import tpu_sc as plsc

_NC = 2
_NS = 16
_NW = _NC * _NS
_LANES = 16
_COLS = 512
_R = 16
_NBUF = 4
_UNROLL = 8


def _make_lut_kernel(n_rows: int):
    rows_per_w = n_rows // _NW
    n_chunks = rows_per_w // _R
    n_rounds = n_chunks // _NBUF
    mesh = plsc.VectorSubcoreMesh(core_axis_name="c", subcore_axis_name="s")

    @functools.partial(
        pl.kernel,
        mesh=mesh,
        compiler_params=pltpu.CompilerParams(needs_layout_passes=False),
        out_type=jax.ShapeDtypeStruct((n_rows, _COLS), jnp.float32),
        scratch_types=(
            [pltpu.VMEM((_LANES,), jnp.float32)]
            + [pltpu.VMEM((_R, _COLS), jnp.int32) for _ in range(_NBUF)]
            + [pltpu.VMEM((_R, _COLS), jnp.float32) for _ in range(_NBUF)]
            + [pltpu.SemaphoreType.DMA for _ in range(2 * _NBUF)]
        ),
    )
    def lut_kernel(idx_hbm, tab_hbm, out_hbm, tab_v, *bufs):
        idx_bufs = bufs[:_NBUF]
        out_bufs = bufs[_NBUF:2 * _NBUF]
        in_sems = bufs[2 * _NBUF:3 * _NBUF]
        out_sems = bufs[3 * _NBUF:4 * _NBUF]

        wid = lax.axis_index("s") * _NC + lax.axis_index("c")
        base = wid * rows_per_w
        pltpu.sync_copy(tab_hbm, tab_v)
        tab_vec = tab_v[...]

        def in_slice(g):
            return idx_hbm.at[pl.ds(base + g * _R, _R)]

        def out_slice(g):
            return out_hbm.at[pl.ds(base + g * _R, _R)]

        def compute(buf_i, buf_o):
            @plsc.parallel_loop(0, _R * _COLS, step=_LANES, unroll=_UNROLL)
            def _(o):
                r = o >> 9
                c = o & (_COLS - 1)
                iv = buf_i[r, pl.ds(c, _LANES)]
                buf_o[r, pl.ds(c, _LANES)] = jnp.take_along_axis(
                    tab_vec, iv, axis=0, mode="promise_in_bounds")

        def slot_step(h, g, buf_i, buf_o, isem, osem):
            pltpu.make_async_copy(in_slice(g), buf_i, isem).wait()

            @pl.when(h > 0)
            def _():
                pltpu.make_async_copy(buf_o, out_slice(g - _NBUF), osem).wait()

            compute(buf_i, buf_o)
            pltpu.async_copy(buf_o, out_slice(g), osem)

            @pl.when(h < n_rounds - 1)
            def _():
                pltpu.async_copy(in_slice(g + _NBUF), buf_i, isem)

        for s in range(_NBUF):
            pltpu.async_copy(in_slice(s), idx_bufs[s], in_sems[s])

        def ring_round(h, carry):
            g0 = _NBUF * h
            for s in range(_NBUF):
                slot_step(h, g0 + s, idx_bufs[s], out_bufs[s],
                          in_sems[s], out_sems[s])
            return carry

        lax.fori_loop(0, n_rounds, ring_round, 0)
        for s in range(_NBUF):
            pltpu.make_async_copy(
                out_bufs[s], out_slice(n_chunks - _NBUF + s),
                out_sems[s]).wait()

    return lut_kernel


def kernel(shortest_path_len, spatial_embeddings):
    B, N, M = shortest_path_len.shape
    n_rows = B * N
    idx2d = shortest_path_len.reshape(n_rows, M).astype(jnp.int32)
    tab16 = jnp.pad(
        spatial_embeddings.reshape(-1).astype(jnp.float32),
        (0, _LANES - spatial_embeddings.shape[0]))
    out2d = _make_lut_kernel(n_rows)(idx2d, tab16)
    return out2d.reshape(B, N, M)
